# baseline (speedup 1.0000x reference)
import functools

import jax
import jax.numpy as jnp
from jax import lax
from jax.experimental import pallas as pl
from jax.experimental.pallas import tpu as pltpu
from jax.experimental.pallas import tpu_sc as plsc

NUM_EMB = 1000000
DIM = 32
BATCH = 16384
HIST = 50
TOTAL = BATCH * HIST

IDX_W = 256
N_ROWS = TOTAL // IDX_W
K = 5


def _gather_sc(flat_idx2d, weight):
    info = plsc.get_sparse_core_info()
    nw = info.num_cores * info.num_subcores
    rows_per_w = N_ROWS // nw
    groups = rows_per_w // K
    mesh = plsc.VectorSubcoreMesh(core_axis_name="c", subcore_axis_name="s")

    @functools.partial(
        pl.kernel,
        out_type=jax.ShapeDtypeStruct((N_ROWS, IDX_W, DIM), jnp.float32),
        mesh=mesh,
        scratch_types=[
            pltpu.VMEM((rows_per_w, IDX_W), jnp.int32),
            pltpu.VMEM((K, IDX_W, DIM), jnp.float32),
            pltpu.VMEM((K, IDX_W, DIM), jnp.float32),
            pltpu.SemaphoreType.DMA,
            pltpu.SemaphoreType.DMA,
            pltpu.SemaphoreType.DMA,
            pltpu.SemaphoreType.DMA,
        ],
        compiler_params=pltpu.CompilerParams(use_tc_tiling_on_sc=False),
    )
    def k(idx_hbm, table_hbm, out_hbm, idx_v, rows0, rows1, gs0, gs1, ws0,
          ws1):
        wid = lax.axis_index("s") * info.num_cores + lax.axis_index("c")
        base = wid * rows_per_w
        rows_v = (rows0, rows1)
        gsem = (gs0, gs1)
        wsem = (ws0, ws1)

        pltpu.sync_copy(idx_hbm.at[pl.ds(base, rows_per_w)], idx_v)

        def fire_gather(g, b):
            for j in range(K):
                pltpu.async_copy(table_hbm.at[idx_v.at[g * K + j]],
                                 rows_v[b].at[j], gsem[b])

        def wait_gather(g, b):
            for j in range(K):
                pltpu.make_async_copy(table_hbm.at[idx_v.at[g * K + j]],
                                      rows_v[b].at[j], gsem[b]).wait()

        def fire_wb(g, b):
            pltpu.async_copy(rows_v[b], out_hbm.at[pl.ds(base + g * K, K)],
                             wsem[b])

        def wait_wb(g, b):
            pltpu.make_async_copy(rows_v[b],
                                  out_hbm.at[pl.ds(base + g * K, K)],
                                  wsem[b]).wait()

        fire_gather(0, 0)

        @pl.loop(0, groups, step=2)
        def _(g):
            @pl.when(g > 0)
            def _():
                wait_wb(g - 1, 1)
            fire_gather(g + 1, 1)
            wait_gather(g, 0)
            fire_wb(g, 0)

            @pl.when(g + 2 < groups)
            def _():
                wait_wb(g, 0)
                fire_gather(g + 2, 0)
            wait_gather(g + 1, 1)
            fire_wb(g + 1, 1)

        wait_wb(groups - 2, 0)
        wait_wb(groups - 1, 1)

    return k(flat_idx2d, weight)


_gather_jit = jax.jit(_gather_sc)


def kernel(indices, weight):
    flat = indices.reshape(N_ROWS, IDX_W)
    out = _gather_jit(flat, weight)
    return out.reshape(BATCH, HIST, DIM)


# speedup vs baseline: 1.3978x; 1.3978x over previous
import functools

import jax
import jax.numpy as jnp
from jax import lax
from jax.experimental import pallas as pl
from jax.experimental.pallas import tpu as pltpu
from jax.experimental.pallas import tpu_sc as plsc

NUM_EMB = 1000000
DIM = 32
BATCH = 16384
HIST = 50
TOTAL = BATCH * HIST

IDX_W = 64
N_ROWS = TOTAL // IDX_W
K = 25


def _gather_sc(flat_idx2d, weight):
    info = plsc.get_sparse_core_info()
    nw = info.num_cores * info.num_subcores
    rows_per_w = N_ROWS // nw
    groups = rows_per_w // K
    mesh = plsc.VectorSubcoreMesh(core_axis_name="c", subcore_axis_name="s")

    @functools.partial(
        pl.kernel,
        out_type=jax.ShapeDtypeStruct((N_ROWS, IDX_W, DIM), jnp.float32),
        mesh=mesh,
        scratch_types=[
            pltpu.VMEM((rows_per_w, IDX_W), jnp.int32),
            pltpu.VMEM((K, IDX_W, DIM), jnp.float32),
            pltpu.VMEM((K, IDX_W, DIM), jnp.float32),
            pltpu.SemaphoreType.DMA,
            pltpu.SemaphoreType.DMA,
            pltpu.SemaphoreType.DMA,
            pltpu.SemaphoreType.DMA,
        ],
        compiler_params=pltpu.CompilerParams(use_tc_tiling_on_sc=False),
    )
    def k(idx_hbm, table_hbm, out_hbm, idx_v, rows0, rows1, gs0, gs1, ws0,
          ws1):
        wid = lax.axis_index("s") * info.num_cores + lax.axis_index("c")
        base = wid * rows_per_w
        rows_v = (rows0, rows1)
        gsem = (gs0, gs1)
        wsem = (ws0, ws1)

        pltpu.sync_copy(idx_hbm.at[pl.ds(base, rows_per_w)], idx_v)

        def fire_gather(g, b):
            for j in range(K):
                pltpu.async_copy(table_hbm.at[idx_v.at[g * K + j]],
                                 rows_v[b].at[j], gsem[b])

        def wait_gather(g, b):
            for j in range(K):
                pltpu.make_async_copy(table_hbm.at[idx_v.at[g * K + j]],
                                      rows_v[b].at[j], gsem[b]).wait()

        def fire_wb(g, b):
            pltpu.async_copy(rows_v[b], out_hbm.at[pl.ds(base + g * K, K)],
                             wsem[b])

        def wait_wb(g, b):
            pltpu.make_async_copy(rows_v[b],
                                  out_hbm.at[pl.ds(base + g * K, K)],
                                  wsem[b]).wait()

        fire_gather(0, 0)

        @pl.loop(0, groups, step=2)
        def _(g):
            @pl.when(g > 0)
            def _():
                wait_wb(g - 1, 1)
            fire_gather(g + 1, 1)
            wait_gather(g, 0)
            fire_wb(g, 0)

            @pl.when(g + 2 < groups)
            def _():
                wait_wb(g, 0)
                fire_gather(g + 2, 0)
            wait_gather(g + 1, 1)
            fire_wb(g + 1, 1)

        wait_wb(groups - 2, 0)
        wait_wb(groups - 1, 1)

    return k(flat_idx2d, weight)


_gather_jit = jax.jit(_gather_sc)


def kernel(indices, weight):
    flat = indices.reshape(N_ROWS, IDX_W)
    out = _gather_jit(flat, weight)
    return out.reshape(BATCH, HIST, DIM)


# speedup vs baseline: 1.4025x; 1.0034x over previous
import functools

import jax
import jax.numpy as jnp
from jax import lax
from jax.experimental import pallas as pl
from jax.experimental.pallas import tpu as pltpu
from jax.experimental.pallas import tpu_sc as plsc

NUM_EMB = 1000000
DIM = 32
BATCH = 16384
HIST = 50
TOTAL = BATCH * HIST

IDX_W = 32
N_ROWS = TOTAL // IDX_W
K = 50


def _gather_sc(flat_idx2d, weight):
    info = plsc.get_sparse_core_info()
    nw = info.num_cores * info.num_subcores
    rows_per_w = N_ROWS // nw
    groups = rows_per_w // K
    mesh = plsc.VectorSubcoreMesh(core_axis_name="c", subcore_axis_name="s")

    @functools.partial(
        pl.kernel,
        out_type=jax.ShapeDtypeStruct((N_ROWS, IDX_W, DIM), jnp.float32),
        mesh=mesh,
        scratch_types=[
            pltpu.VMEM((rows_per_w, IDX_W), jnp.int32),
            pltpu.VMEM((K, IDX_W, DIM), jnp.float32),
            pltpu.VMEM((K, IDX_W, DIM), jnp.float32),
            pltpu.SemaphoreType.DMA,
            pltpu.SemaphoreType.DMA,
            pltpu.SemaphoreType.DMA,
            pltpu.SemaphoreType.DMA,
        ],
        compiler_params=pltpu.CompilerParams(use_tc_tiling_on_sc=False),
    )
    def k(idx_hbm, table_hbm, out_hbm, idx_v, rows0, rows1, gs0, gs1, ws0,
          ws1):
        wid = lax.axis_index("s") * info.num_cores + lax.axis_index("c")
        base = wid * rows_per_w
        rows_v = (rows0, rows1)
        gsem = (gs0, gs1)
        wsem = (ws0, ws1)

        pltpu.sync_copy(idx_hbm.at[pl.ds(base, rows_per_w)], idx_v)

        def fire_gather(g, b):
            for j in range(K):
                pltpu.async_copy(table_hbm.at[idx_v.at[g * K + j]],
                                 rows_v[b].at[j], gsem[b])

        def wait_gather(g, b):
            for j in range(K):
                pltpu.make_async_copy(table_hbm.at[idx_v.at[g * K + j]],
                                      rows_v[b].at[j], gsem[b]).wait()

        def fire_wb(g, b):
            pltpu.async_copy(rows_v[b], out_hbm.at[pl.ds(base + g * K, K)],
                             wsem[b])

        def wait_wb(g, b):
            pltpu.make_async_copy(rows_v[b],
                                  out_hbm.at[pl.ds(base + g * K, K)],
                                  wsem[b]).wait()

        fire_gather(0, 0)

        @pl.loop(0, groups, step=2)
        def _(g):
            @pl.when(g > 0)
            def _():
                wait_wb(g - 1, 1)
            fire_gather(g + 1, 1)
            wait_gather(g, 0)
            fire_wb(g, 0)

            @pl.when(g + 2 < groups)
            def _():
                wait_wb(g, 0)
                fire_gather(g + 2, 0)
            wait_gather(g + 1, 1)
            fire_wb(g + 1, 1)

        wait_wb(groups - 2, 0)
        wait_wb(groups - 1, 1)

    return k(flat_idx2d, weight)


_gather_jit = jax.jit(_gather_sc)


def kernel(indices, weight):
    flat = indices.reshape(N_ROWS, IDX_W)
    out = _gather_jit(flat, weight)
    return out.reshape(BATCH, HIST, DIM)


# speedup vs baseline: 1.4987x; 1.0686x over previous
import functools

import jax
import jax.numpy as jnp
from jax import lax
from jax.experimental import pallas as pl
from jax.experimental.pallas import tpu as pltpu
from jax.experimental.pallas import tpu_sc as plsc

NUM_EMB = 1000000
DIM = 32
BATCH = 16384
HIST = 50
TOTAL = BATCH * HIST

IDX_W = 32
N_ROWS = TOTAL // IDX_W
K = 50


def _gather_sc(flat_idx2d, weight):
    info = plsc.get_sparse_core_info()
    nw = info.num_cores * info.num_subcores
    rows_per_w = N_ROWS // nw
    groups = rows_per_w // K
    mesh = plsc.VectorSubcoreMesh(core_axis_name="c", subcore_axis_name="s")

    @functools.partial(
        pl.kernel,
        out_type=jax.ShapeDtypeStruct((N_ROWS, IDX_W, DIM), jnp.float32),
        mesh=mesh,
        scratch_types=[
            pltpu.VMEM((rows_per_w, IDX_W), jnp.int32),
            pltpu.VMEM((K, IDX_W, DIM), jnp.float32),
            pltpu.VMEM((K, IDX_W, DIM), jnp.float32),
            pltpu.SemaphoreType.DMA,
            pltpu.SemaphoreType.DMA,
            pltpu.SemaphoreType.DMA,
            pltpu.SemaphoreType.DMA,
        ],
        compiler_params=pltpu.CompilerParams(use_tc_tiling_on_sc=False),
    )
    def k(idx_hbm, table_hbm, out_hbm, idx_v, rows0, rows1, gs0, gs1, ws0,
          ws1):
        wid = lax.axis_index("s") * info.num_cores + lax.axis_index("c")
        base = wid * rows_per_w
        rows_v = (rows0, rows1)
        gsem = (gs0, gs1)
        wsem = (ws0, ws1)

        pltpu.sync_copy(idx_hbm.at[pl.ds(base, rows_per_w)], idx_v)

        def fire_gather(g, b):
            for j in range(K):
                pltpu.async_copy(table_hbm.at[idx_v.at[g * K + j]],
                                 rows_v[b].at[j], gsem[b])

        def wait_gather(g, b):
            for j in range(K):
                pltpu.make_async_copy(table_hbm.at[idx_v.at[g * K + j]],
                                      rows_v[b].at[j], gsem[b]).wait()

        def fire_wb(g, b):
            pltpu.async_copy(rows_v[b], out_hbm.at[pl.ds(base + g * K, K)],
                             wsem[b])

        def wait_wb(g, b):
            pltpu.make_async_copy(rows_v[b],
                                  out_hbm.at[pl.ds(base + g * K, K)],
                                  wsem[b]).wait()

        fire_gather(0, 0)

        @pl.loop(0, groups, step=2)
        def _(g):
            @pl.when(g > 0)
            def _():
                wait_wb(g - 1, 1)
            fire_gather(g + 1, 1)
            wait_gather(g, 0)
            fire_wb(g, 0)

            @pl.when(g + 2 < groups)
            def _():
                wait_wb(g, 0)
                fire_gather(g + 2, 0)
            wait_gather(g + 1, 1)
            fire_wb(g + 1, 1)

        wait_wb(groups - 2, 0)
        wait_wb(groups - 1, 1)

    return k(flat_idx2d, weight)


_gather_jit = jax.jit(_gather_sc)


def kernel(indices, weight):
    flat = indices.T.reshape(N_ROWS, IDX_W)
    out = _gather_jit(flat, weight)
    return out.reshape(HIST, BATCH, DIM).transpose(1, 0, 2)
